# Initial kernel scaffold; baseline (speedup 1.0000x reference)
import jax
import jax.numpy as jnp
from jax.experimental import pallas as pl


def kernel(X, y, edge_index, edge_weight, skip, params):
    raise NotImplementedError("write your pallas kernel here")


# speedup vs baseline: 7.0101x; 7.0101x over previous
import functools

import jax
import jax.numpy as jnp
from jax.experimental import pallas as pl
from jax.experimental.pallas import tpu as pltpu

_F = 4
_HID = 32
_W = 48
_GATES = 4 * _HID
_BLK = 2048


def _cell_body(qu_ref, c_ref, wall_ref, bcat_ref, lng_ref, lnb_ref,
               wfc_ref, bfc_ref, h_out, c_out, p_out):
    q = qu_ref[...]
    p = jnp.dot(q, wall_ref[...], preferred_element_type=jnp.float32)
    p = p + bcat_ref[...]
    i = jax.nn.sigmoid(p[:, 0 * _HID:1 * _HID])
    f = jax.nn.sigmoid(p[:, 1 * _HID:2 * _HID])
    g = jnp.tanh(p[:, 2 * _HID:3 * _HID])
    o = jax.nn.sigmoid(p[:, 3 * _HID:4 * _HID])
    c_new = f * c_ref[...] + i * g
    h_new = o * jnp.tanh(c_new)
    h_out[...] = h_new
    c_out[...] = c_new
    out = jax.nn.relu(h_new)
    mu = jnp.mean(out, axis=1, keepdims=True)
    var = jnp.mean((out - mu) * (out - mu), axis=1, keepdims=True)
    outn = (out - mu) * jax.lax.rsqrt(var + 1e-5) * lng_ref[...] + lnb_ref[...]
    pred = jnp.sum(outn * wfc_ref[...], axis=1, keepdims=True) + bfc_ref[...]
    p_out[...] = jax.nn.sigmoid(pred)


@functools.partial(jax.jit, static_argnames=("n",))
def _cell_call(qu, c, wall, bcat, lng, lnb, wfc, bfc, *, n):
    grid = (pl.cdiv(n, _BLK),)
    row = lambda i: (i, 0)
    fix = lambda i: (0, 0)
    return pl.pallas_call(
        _cell_body,
        grid=grid,
        in_specs=[
            pl.BlockSpec((_BLK, _W), row),
            pl.BlockSpec((_BLK, _HID), row),
            pl.BlockSpec((_W, _GATES), fix),
            pl.BlockSpec((1, _GATES), fix),
            pl.BlockSpec((1, _HID), fix),
            pl.BlockSpec((1, _HID), fix),
            pl.BlockSpec((1, _HID), fix),
            pl.BlockSpec((1, 1), fix),
        ],
        out_specs=[
            pl.BlockSpec((_BLK, _HID), row),
            pl.BlockSpec((_BLK, _HID), row),
            pl.BlockSpec((_BLK, 1), row),
        ],
        out_shape=[
            jax.ShapeDtypeStruct((n, _HID), jnp.float32),
            jax.ShapeDtypeStruct((n, _HID), jnp.float32),
            jax.ShapeDtypeStruct((n, 1), jnp.float32),
        ],
    )(qu, c, wall, bcat, lng, lnb, wfc, bfc)


def kernel(X, y, edge_index, edge_weight, skip, params):
    n = X.shape[1]
    tin = X.shape[0]
    tout = y.shape[0]
    src = edge_index[0]
    dst = edge_index[1]

    wxcat = jnp.concatenate(
        [params['Wx_i'], params['Wx_f'], params['Wx_g'], params['Wx_o']], axis=1)
    whcat = jnp.concatenate(
        [params['Wh_i'], params['Wh_f'], params['Wh_g'], params['Wh_o']], axis=1)
    wall = jnp.zeros((_W, _GATES), jnp.float32)
    wall = wall.at[0:_F].set(wxcat).at[_F:_F + _HID].set(whcat)
    bcat = jnp.concatenate(
        [params['b_i'], params['b_f'], params['b_g'], params['b_o']])[None, :]
    lng = params['ln_g'][None, :]
    lnb = params['ln_b'][None, :]
    wfc = params['Wfc'][:, 0][None, :]
    bfc = params['bfc'][None, :]

    def propagate(u):
        msg = u[src] * edge_weight[:, None]
        return jax.ops.segment_sum(msg, dst, num_segments=n)

    def step(u, c):
        return _cell_call(propagate(u), c, wall, bcat, lng, lnb, wfc, bfc, n=n)

    u = jnp.zeros((n, _W), jnp.float32).at[:, 0:_F].set(X[tin - 1])
    h, c, _ = step(u, jnp.zeros((n, _HID), jnp.float32))

    x = jnp.zeros((n, _F), jnp.float32)
    outs = []
    for _ in range(tout):
        u = jnp.zeros((n, _W), jnp.float32)
        u = u.at[:, 0:_F].set(x).at[:, _F:_F + _HID].set(h)
        h, c, pred = step(u, c)
        outs.append(pred)
        x = jnp.concatenate([x[:, 1:], pred], axis=1)
    return jnp.stack(outs)


# speedup vs baseline: 23.5034x; 3.3528x over previous
import functools

import jax
import jax.numpy as jnp
from jax import lax
from jax.experimental import pallas as pl
from jax.experimental.pallas import tpu as pltpu
from jax.experimental.pallas import tpu_sc as plsc

_F = 4
_HID = 32
_W = 48
_GATES = 4 * _HID
_BLK = 2048

_N = 100000
_E = 1600000
_NW = 32
_EPAD = 1638400
_EPW = _EPAD // _NW
_KB = 3200
_NBLK = _EPW // _KB
_CH = 128
_NCH = _KB // _CH
_SLOTS = 8
_CS = 16
_NSL = _W // _CS
_NPAD = 100096
_RCH = _NPAD // 16


def _prop_body(u0_hbm, u1_hbm, u2_hbm, src_hbm, dst_hbm, w_hbm, z_hbm,
               out_hbm, sbuf, dbuf, wbuf, stage, rows, acc, sems):
    cidx = lax.axis_index("c")
    sidx = lax.axis_index("s")
    wid = cidx * 16 + sidx
    ebase = wid * _EPW
    u_slices = (u0_hbm, u1_hbm, u2_hbm)

    for r in range(_NSL):
        u_hbm = u_slices[r]
        pltpu.sync_copy(z_hbm.at[pl.ds(sidx * _RCH, _RCH)],
                        acc.at[pl.ds(sidx * _RCH, _RCH)])
        plsc.subcore_barrier()

        def block_body(b, _, u_hbm=u_hbm):
            base = ebase + b * _KB
            pltpu.sync_copy(src_hbm.at[pl.ds(base, _KB)], sbuf)
            pltpu.sync_copy(dst_hbm.at[pl.ds(base, _KB)], dbuf)
            pltpu.sync_copy(w_hbm.at[pl.ds(base, _KB)], wbuf)

            def abody(a, _, u_hbm=u_hbm):
                for t in range(_SLOTS):
                    j = a * _SLOTS + t

                    @pl.when(j < _NCH)
                    def _(j=j, t=t):
                        pltpu.async_copy(
                            u_hbm.at[sbuf.at[pl.ds(j * _CH, _CH)]],
                            rows.at[t], sems.at[t])

                for t in range(_SLOTS):
                    j = a * _SLOTS + t

                    @pl.when(j < _NCH)
                    def _(j=j, t=t):
                        pltpu.make_async_copy(
                            u_hbm.at[sbuf.at[pl.ds(j * _CH, _CH)]],
                            rows.at[t], sems.at[t]).wait()
                        rt = rows.at[t]

                        def gbody(g, _, j=j, rt=rt):
                            wv = wbuf[pl.ds(j * _CH + g * 16, 16)]
                            for jj in range(16):
                                e = g * 16 + jj
                                rt[e] = rt[e] * wv[jj]
                            stage[0, pl.ds(g * 16, 16)] = dbuf[pl.ds(j * _CH + g * 16, 16)]
                            return 0

                        lax.fori_loop(0, _CH // 16, gbody, 0)
                        pltpu.sync_copy(rt, acc.at[stage.at[0]], add=True)

                return 0

            lax.fori_loop(0, (_NCH + _SLOTS - 1) // _SLOTS, abody, 0)
            return 0

        lax.fori_loop(0, _NBLK, block_body, 0)
        plsc.subcore_barrier()
        pltpu.sync_copy(acc.at[pl.ds(sidx * _RCH, _RCH)],
                        out_hbm.at[cidx, r, pl.ds(sidx * _RCH, _RCH)])
        plsc.subcore_barrier()


_propagate_sc = functools.partial(
    pl.kernel,
    out_type=jax.ShapeDtypeStruct((2, _NSL, _NPAD, _CS), jnp.float32),
    compiler_params=pltpu.CompilerParams(use_tc_tiling_on_sc=False),
    mesh=plsc.VectorSubcoreMesh(core_axis_name="c", subcore_axis_name="s"),
    scratch_types=[
        pltpu.VMEM((_KB,), jnp.int32),
        pltpu.VMEM((_KB,), jnp.int32),
        pltpu.VMEM((_KB,), jnp.float32),
        pltpu.VMEM((1, _CH), jnp.int32),
        pltpu.VMEM((_SLOTS, _CH, _CS), jnp.float32),
        pltpu.VMEM_SHARED((_NPAD, _CS), jnp.float32),
        pltpu.SemaphoreType.DMA((_SLOTS,)),
    ],
)(_prop_body)


def _cell_body(qu_ref, c_ref, wall_ref, bcat_ref, lng_ref, lnb_ref,
               wfc_ref, bfc_ref, h_out, c_out, p_out):
    q = qu_ref[...]
    p = jnp.dot(q, wall_ref[...], preferred_element_type=jnp.float32)
    p = p + bcat_ref[...]
    i = jax.nn.sigmoid(p[:, 0 * _HID:1 * _HID])
    f = jax.nn.sigmoid(p[:, 1 * _HID:2 * _HID])
    g = jnp.tanh(p[:, 2 * _HID:3 * _HID])
    o = jax.nn.sigmoid(p[:, 3 * _HID:4 * _HID])
    c_new = f * c_ref[...] + i * g
    h_new = o * jnp.tanh(c_new)
    h_out[...] = h_new
    c_out[...] = c_new
    out = jax.nn.relu(h_new)
    mu = jnp.mean(out, axis=1, keepdims=True)
    var = jnp.mean((out - mu) * (out - mu), axis=1, keepdims=True)
    outn = (out - mu) * jax.lax.rsqrt(var + 1e-5) * lng_ref[...] + lnb_ref[...]
    pred = jnp.sum(outn * wfc_ref[...], axis=1, keepdims=True) + bfc_ref[...]
    p_out[...] = jax.nn.sigmoid(pred)


@functools.partial(jax.jit, static_argnames=("n",))
def _cell_call(qu, c, wall, bcat, lng, lnb, wfc, bfc, *, n):
    grid = (pl.cdiv(n, _BLK),)
    row = lambda i: (i, 0)
    fix = lambda i: (0, 0)
    return pl.pallas_call(
        _cell_body,
        grid=grid,
        in_specs=[
            pl.BlockSpec((_BLK, _W), row),
            pl.BlockSpec((_BLK, _HID), row),
            pl.BlockSpec((_W, _GATES), fix),
            pl.BlockSpec((1, _GATES), fix),
            pl.BlockSpec((1, _HID), fix),
            pl.BlockSpec((1, _HID), fix),
            pl.BlockSpec((1, _HID), fix),
            pl.BlockSpec((1, 1), fix),
        ],
        out_specs=[
            pl.BlockSpec((_BLK, _HID), row),
            pl.BlockSpec((_BLK, _HID), row),
            pl.BlockSpec((_BLK, 1), row),
        ],
        out_shape=[
            jax.ShapeDtypeStruct((n, _HID), jnp.float32),
            jax.ShapeDtypeStruct((n, _HID), jnp.float32),
            jax.ShapeDtypeStruct((n, 1), jnp.float32),
        ],
    )(qu, c, wall, bcat, lng, lnb, wfc, bfc)


def kernel(X, y, edge_index, edge_weight, skip, params):
    n = X.shape[1]
    tin = X.shape[0]
    tout = y.shape[0]
    epad = _EPAD - edge_index.shape[1]
    src = jnp.concatenate([edge_index[0], jnp.zeros((epad,), jnp.int32)])
    dst = jnp.concatenate([edge_index[1], jnp.zeros((epad,), jnp.int32)])
    wgt = jnp.concatenate([edge_weight, jnp.zeros((epad,), jnp.float32)])

    wxcat = jnp.concatenate(
        [params['Wx_i'], params['Wx_f'], params['Wx_g'], params['Wx_o']], axis=1)
    whcat = jnp.concatenate(
        [params['Wh_i'], params['Wh_f'], params['Wh_g'], params['Wh_o']], axis=1)
    wall = jnp.zeros((_W, _GATES), jnp.float32)
    wall = wall.at[0:_F].set(wxcat).at[_F:_F + _HID].set(whcat)
    bcat = jnp.concatenate(
        [params['b_i'], params['b_f'], params['b_g'], params['b_o']])[None, :]
    lng = params['ln_g'][None, :]
    lnb = params['ln_b'][None, :]
    wfc = params['Wfc'][:, 0][None, :]
    bfc = params['bfc'][None, :]

    zeros_acc = jnp.zeros((_NPAD, _CS), jnp.float32)

    def step(u, c):
        parts = _propagate_sc(u[:, 0:_CS], u[:, _CS:2 * _CS], u[:, 2 * _CS:3 * _CS],
                              src, dst, wgt, zeros_acc)
        qu = (parts[0] + parts[1]).transpose(1, 0, 2).reshape(_NPAD, _W)
        return _cell_call(qu, c, wall, bcat, lng, lnb, wfc, bfc, n=_NPAD)

    u = jnp.zeros((_NPAD, _W), jnp.float32).at[:n, 0:_F].set(X[tin - 1])
    h, c, _ = step(u, jnp.zeros((_NPAD, _HID), jnp.float32))

    x = jnp.zeros((_NPAD, _F), jnp.float32)
    outs = []
    for _ in range(tout):
        u = jnp.zeros((_NPAD, _W), jnp.float32)
        u = u.at[:, 0:_F].set(x).at[:, _F:_F + _HID].set(h)
        h, c, pred = step(u, c)
        outs.append(pred[:n])
        x = jnp.concatenate([x[:, 1:], pred], axis=1)
    return jnp.stack(outs)


# speedup vs baseline: 24.1633x; 1.0281x over previous
import functools

import jax
import jax.numpy as jnp
from jax import lax
from jax.experimental import pallas as pl
from jax.experimental.pallas import tpu as pltpu
from jax.experimental.pallas import tpu_sc as plsc

_F = 4
_HID = 32
_W = 48
_GATES = 4 * _HID
_BLK = 2048

_N = 100000
_E = 1600000
_NW = 32
_EPAD = 1638400
_EPW = _EPAD // _NW
_KB = 3200
_NBLK = _EPW // _KB
_CH = 128
_NCH = _KB // _CH
_SLOTS = 8
_CS = 16
_NSL = _W // _CS
_NPAD = 100096
_RCH = _NPAD // 16


def _prop_body(u0_hbm, u1_hbm, u2_hbm, src_hbm, dst_hbm, w_hbm, z_hbm,
               out_hbm, sbuf, dbuf, wbuf, stage, rows, acc, sems, ssems):
    cidx = lax.axis_index("c")
    sidx = lax.axis_index("s")
    wid = cidx * 16 + sidx
    ebase = wid * _EPW
    u_slices = (u0_hbm, u1_hbm, u2_hbm)

    for r in range(_NSL):
        u_hbm = u_slices[r]
        pltpu.sync_copy(z_hbm.at[pl.ds(sidx * _RCH, _RCH)],
                        acc.at[pl.ds(sidx * _RCH, _RCH)])
        plsc.subcore_barrier()

        def block_body(b, _, u_hbm=u_hbm):
            base = ebase + b * _KB
            pltpu.sync_copy(src_hbm.at[pl.ds(base, _KB)], sbuf)
            pltpu.sync_copy(dst_hbm.at[pl.ds(base, _KB)], dbuf)
            pltpu.sync_copy(w_hbm.at[pl.ds(base, _KB)], wbuf)

            def abody(a, _, u_hbm=u_hbm):
                for t in range(_SLOTS):
                    j = a * _SLOTS + t

                    @pl.when(j < _NCH)
                    def _(j=j, t=t):
                        pltpu.async_copy(
                            u_hbm.at[sbuf.at[pl.ds(j * _CH, _CH)]],
                            rows.at[t], sems.at[t])

                for t in range(_SLOTS):
                    j = a * _SLOTS + t

                    @pl.when(j < _NCH)
                    def _(j=j, t=t):
                        pltpu.make_async_copy(
                            u_hbm.at[sbuf.at[pl.ds(j * _CH, _CH)]],
                            rows.at[t], sems.at[t]).wait()
                        rt = rows.at[t]

                        def gbody(g, _, j=j, t=t, rt=rt):
                            wv = wbuf[pl.ds(j * _CH + g * 16, 16)]
                            for jj in range(16):
                                e = g * 16 + jj
                                rt[e] = rt[e] * wv[jj]
                            stage[t, 0, pl.ds(g * 16, 16)] = dbuf[pl.ds(j * _CH + g * 16, 16)]
                            return 0

                        lax.fori_loop(0, _CH // 16, gbody, 0)
                        pltpu.async_copy(rt, acc.at[stage.at[t, 0]], ssems.at[t],
                                         add=True)

                for t in range(_SLOTS):
                    j = a * _SLOTS + t

                    @pl.when(j < _NCH)
                    def _(j=j, t=t):
                        pltpu.make_async_copy(
                            rows.at[t], acc.at[stage.at[t, 0]], ssems.at[t]).wait()

                return 0

            lax.fori_loop(0, (_NCH + _SLOTS - 1) // _SLOTS, abody, 0)
            return 0

        lax.fori_loop(0, _NBLK, block_body, 0)
        plsc.subcore_barrier()
        pltpu.sync_copy(acc.at[pl.ds(sidx * _RCH, _RCH)],
                        out_hbm.at[cidx, r, pl.ds(sidx * _RCH, _RCH)])
        plsc.subcore_barrier()


_propagate_sc = functools.partial(
    pl.kernel,
    out_type=jax.ShapeDtypeStruct((2, _NSL, _NPAD, _CS), jnp.float32),
    compiler_params=pltpu.CompilerParams(use_tc_tiling_on_sc=False),
    mesh=plsc.VectorSubcoreMesh(core_axis_name="c", subcore_axis_name="s"),
    scratch_types=[
        pltpu.VMEM((_KB,), jnp.int32),
        pltpu.VMEM((_KB,), jnp.int32),
        pltpu.VMEM((_KB,), jnp.float32),
        pltpu.VMEM((_SLOTS, 1, _CH), jnp.int32),
        pltpu.VMEM((_SLOTS, _CH, _CS), jnp.float32),
        pltpu.VMEM_SHARED((_NPAD, _CS), jnp.float32),
        pltpu.SemaphoreType.DMA((_SLOTS,)),
        pltpu.SemaphoreType.DMA((_SLOTS,)),
    ],
)(_prop_body)


def _cell_body(qu_ref, c_ref, wall_ref, bcat_ref, lng_ref, lnb_ref,
               wfc_ref, bfc_ref, h_out, c_out, p_out):
    q = qu_ref[...]
    p = jnp.dot(q, wall_ref[...], preferred_element_type=jnp.float32)
    p = p + bcat_ref[...]
    i = jax.nn.sigmoid(p[:, 0 * _HID:1 * _HID])
    f = jax.nn.sigmoid(p[:, 1 * _HID:2 * _HID])
    g = jnp.tanh(p[:, 2 * _HID:3 * _HID])
    o = jax.nn.sigmoid(p[:, 3 * _HID:4 * _HID])
    c_new = f * c_ref[...] + i * g
    h_new = o * jnp.tanh(c_new)
    h_out[...] = h_new
    c_out[...] = c_new
    out = jax.nn.relu(h_new)
    mu = jnp.mean(out, axis=1, keepdims=True)
    var = jnp.mean((out - mu) * (out - mu), axis=1, keepdims=True)
    outn = (out - mu) * jax.lax.rsqrt(var + 1e-5) * lng_ref[...] + lnb_ref[...]
    pred = jnp.sum(outn * wfc_ref[...], axis=1, keepdims=True) + bfc_ref[...]
    p_out[...] = jax.nn.sigmoid(pred)


@functools.partial(jax.jit, static_argnames=("n",))
def _cell_call(qu, c, wall, bcat, lng, lnb, wfc, bfc, *, n):
    grid = (pl.cdiv(n, _BLK),)
    row = lambda i: (i, 0)
    fix = lambda i: (0, 0)
    return pl.pallas_call(
        _cell_body,
        grid=grid,
        in_specs=[
            pl.BlockSpec((_BLK, _W), row),
            pl.BlockSpec((_BLK, _HID), row),
            pl.BlockSpec((_W, _GATES), fix),
            pl.BlockSpec((1, _GATES), fix),
            pl.BlockSpec((1, _HID), fix),
            pl.BlockSpec((1, _HID), fix),
            pl.BlockSpec((1, _HID), fix),
            pl.BlockSpec((1, 1), fix),
        ],
        out_specs=[
            pl.BlockSpec((_BLK, _HID), row),
            pl.BlockSpec((_BLK, _HID), row),
            pl.BlockSpec((_BLK, 1), row),
        ],
        out_shape=[
            jax.ShapeDtypeStruct((n, _HID), jnp.float32),
            jax.ShapeDtypeStruct((n, _HID), jnp.float32),
            jax.ShapeDtypeStruct((n, 1), jnp.float32),
        ],
    )(qu, c, wall, bcat, lng, lnb, wfc, bfc)


def kernel(X, y, edge_index, edge_weight, skip, params):
    n = X.shape[1]
    tin = X.shape[0]
    tout = y.shape[0]
    epad = _EPAD - edge_index.shape[1]
    src = jnp.concatenate([edge_index[0], jnp.zeros((epad,), jnp.int32)])
    dst = jnp.concatenate([edge_index[1], jnp.zeros((epad,), jnp.int32)])
    wgt = jnp.concatenate([edge_weight, jnp.zeros((epad,), jnp.float32)])

    wxcat = jnp.concatenate(
        [params['Wx_i'], params['Wx_f'], params['Wx_g'], params['Wx_o']], axis=1)
    whcat = jnp.concatenate(
        [params['Wh_i'], params['Wh_f'], params['Wh_g'], params['Wh_o']], axis=1)
    wall = jnp.zeros((_W, _GATES), jnp.float32)
    wall = wall.at[0:_F].set(wxcat).at[_F:_F + _HID].set(whcat)
    bcat = jnp.concatenate(
        [params['b_i'], params['b_f'], params['b_g'], params['b_o']])[None, :]
    lng = params['ln_g'][None, :]
    lnb = params['ln_b'][None, :]
    wfc = params['Wfc'][:, 0][None, :]
    bfc = params['bfc'][None, :]

    zeros_acc = jnp.zeros((_NPAD, _CS), jnp.float32)

    def step(u, c):
        parts = _propagate_sc(u[:, 0:_CS], u[:, _CS:2 * _CS], u[:, 2 * _CS:3 * _CS],
                              src, dst, wgt, zeros_acc)
        qu = (parts[0] + parts[1]).transpose(1, 0, 2).reshape(_NPAD, _W)
        return _cell_call(qu, c, wall, bcat, lng, lnb, wfc, bfc, n=_NPAD)

    u = jnp.zeros((_NPAD, _W), jnp.float32).at[:n, 0:_F].set(X[tin - 1])
    h, c, _ = step(u, jnp.zeros((_NPAD, _HID), jnp.float32))

    x = jnp.zeros((_NPAD, _F), jnp.float32)
    outs = []
    for _ in range(tout):
        u = jnp.zeros((_NPAD, _W), jnp.float32)
        u = u.at[:, 0:_F].set(x).at[:, _F:_F + _HID].set(h)
        h, c, pred = step(u, c)
        outs.append(pred[:n])
        x = jnp.concatenate([x[:, 1:], pred], axis=1)
    return jnp.stack(outs)
